# baseline (speedup 1.0000x reference)
import functools

import jax
import jax.numpy as jnp
from jax import lax
from jax.experimental import pallas as pl
from jax.experimental.pallas import tpu as pltpu
from jax.experimental.pallas import tpu_sc as plsc

N_CATS = 8
EPS = 1e-07

_NC = 2
_NS = 16
_NW = _NC * _NS
_L = 16

_CHUNK_G = 256


def _sc_partial_hist(n_rows, interpret=False):
    assert n_rows % _L == 0
    total_groups = n_rows // _L
    per_worker = total_groups // _NW
    tail_rem = total_groups - per_worker * _NW
    n_full = per_worker // _CHUNK_G
    rem_g = per_worker - n_full * _CHUNK_G
    chunk_rows = _CHUNK_G * _L

    mesh = plsc.VectorSubcoreMesh(
        core_axis_name="c", subcore_axis_name="s",
        num_cores=_NC, num_subcores=_NS)

    @functools.partial(
        pl.kernel,
        out_type=jax.ShapeDtypeStruct((_NW, 64), jnp.int32),
        mesh=mesh,
        interpret=interpret,
        compiler_params=pltpu.CompilerParams(needs_layout_passes=False),
        scratch_types=[
            pltpu.VMEM((chunk_rows * N_CATS,), jnp.float32),
            pltpu.VMEM((chunk_rows,), jnp.int32),
            pltpu.VMEM((_L * 64,), jnp.int32),
            pltpu.VMEM((64,), jnp.int32),
        ],
    )
    def sc_kernel(inp_hbm, tgt_hbm, out_hbm, inp_v, tgt_v, hist_v, part_v):
        wid = lax.axis_index("s") * _NC + lax.axis_index("c")
        iota = lax.iota(jnp.int32, _L)
        lane_off = iota * 64
        iota8 = iota * N_CATS
        jcols = [jnp.full((_L,), j, jnp.int32) for j in range(N_CATS)]
        ones = jnp.full((_L,), 1, jnp.int32)
        zeros = jnp.full((_L,), 0, jnp.int32)

        for k in range(64):
            hist_v[pl.ds(k * _L, _L)] = zeros

        def process_groups(n_groups):
            def body(g, _):
                r0 = g * _L
                flat0 = r0 * N_CATS + iota8
                cols = [plsc.load_gather(inp_v, [flat0 + j])
                        for j in range(N_CATS)]
                bv, bi = [], []
                for j in range(0, N_CATS, 2):
                    gt = cols[j + 1] > cols[j]
                    bv.append(jnp.where(gt, cols[j + 1], cols[j]))
                    bi.append(jnp.where(gt, jcols[j + 1], jcols[j]))
                v0, i0, v1, i1 = bv[0], bi[0], bv[1], bi[1]
                gt = v1 > v0
                va, ia = jnp.where(gt, v1, v0), jnp.where(gt, i1, i0)
                v0, i0, v1, i1 = bv[2], bi[2], bv[3], bi[3]
                gt = v1 > v0
                vb, ib = jnp.where(gt, v1, v0), jnp.where(gt, i1, i0)
                gt = vb > va
                pred = jnp.where(gt, ib, ia)
                tgt = tgt_v[pl.ds(r0, _L)]
                bins = tgt * N_CATS + pred + lane_off
                plsc.addupdate_scatter(hist_v, [bins], ones)
                return 0

            lax.fori_loop(0, n_groups, body, 0)

        row_base = wid * (per_worker * _L)
        for c in range(n_full):
            rbase = row_base + c * chunk_rows
            pltpu.sync_copy(
                inp_hbm.at[pl.ds(rbase * N_CATS, chunk_rows * N_CATS)],
                inp_v)
            pltpu.sync_copy(tgt_hbm.at[pl.ds(rbase, chunk_rows)], tgt_v)
            process_groups(_CHUNK_G)
        if rem_g > 0:
            rbase = row_base + n_full * chunk_rows
            rrows = rem_g * _L
            pltpu.sync_copy(inp_hbm.at[pl.ds(rbase * N_CATS, rrows * N_CATS)],
                            inp_v.at[pl.ds(0, rrows * N_CATS)])
            pltpu.sync_copy(tgt_hbm.at[pl.ds(rbase, rrows)],
                            tgt_v.at[pl.ds(0, rrows)])
            process_groups(rem_g)
        if tail_rem > 0:
            @pl.when(wid < tail_rem)
            def _():
                tbase = _NW * per_worker * _L + wid * _L
                pltpu.sync_copy(inp_hbm.at[pl.ds(tbase * N_CATS, _L * N_CATS)],
                                inp_v.at[pl.ds(0, _L * N_CATS)])
                pltpu.sync_copy(tgt_hbm.at[pl.ds(tbase, _L)],
                                tgt_v.at[pl.ds(0, _L)])
                process_groups(1)

        for q in range(4):
            acc = zeros
            for r in range(_L):
                acc = acc + hist_v[pl.ds(r * 64 + q * _L, _L)]
            part_v[pl.ds(q * _L, _L)] = acc
        pltpu.sync_copy(part_v, out_hbm.at[wid])

    return sc_kernel


def _qwk_epilogue(parts_ref, o_ref, *, n):
    x = parts_ref[...].astype(jnp.float32)
    conf = jnp.sum(x, axis=0) * (1.0 / n)
    marg_true = jnp.sum(conf, axis=1, keepdims=True)
    marg_pred = jnp.sum(conf, axis=0, keepdims=True)
    expected = marg_true * marg_pred
    i = lax.broadcasted_iota(jnp.int32, (N_CATS, N_CATS), 0).astype(jnp.float32)
    j = lax.broadcasted_iota(jnp.int32, (N_CATS, N_CATS), 1).astype(jnp.float32)
    w = 1.0 - (i - j) ** 2 / float((N_CATS - 1) ** 2)
    po = jnp.sum(w * conf)
    pe = jnp.sum(w * expected)
    pe = jnp.clip(pe, 0.0, 1.0 - EPS)
    qwk = jnp.where(pe >= 1.0 - EPS, 0.0, (po - pe) / (1.0 - pe + EPS))
    qwk = jnp.clip(qwk, -1.0, 1.0)
    o_ref[...] = jnp.full((1, 1), 1.0 - qwk, jnp.float32)


def kernel(inputs, targets):
    if inputs.ndim > 2:
        inputs = inputs.reshape(-1, inputs.shape[-1])
        targets = targets.reshape(-1)
    n = inputs.shape[0]
    targets = targets.astype(jnp.int32)
    parts = _sc_partial_hist(n)(inputs.reshape(-1), targets)
    out = pl.pallas_call(
        functools.partial(_qwk_epilogue, n=n),
        out_shape=jax.ShapeDtypeStruct((1, 1), jnp.float32),
    )(parts.reshape(_NW, N_CATS, N_CATS))
    return out[0, 0]


# speedup vs baseline: 1.0596x; 1.0596x over previous
import functools

import jax
import jax.numpy as jnp
from jax import lax
from jax.experimental import pallas as pl
from jax.experimental.pallas import tpu as pltpu
from jax.experimental.pallas import tpu_sc as plsc

N_CATS = 8
EPS = 1e-07

_NC = 2
_NS = 16
_NW = _NC * _NS
_L = 16

def _chunk_groups(per_worker):
    best = 1
    for d in range(1, per_worker + 1):
        if per_worker % d == 0 and d * _L * 36 <= 320_000:
            best = d
    return best


def _sc_partial_hist(n_rows, interpret=False):
    assert n_rows % _L == 0
    total_groups = n_rows // _L
    per_worker = total_groups // _NW
    tail_rem = total_groups - per_worker * _NW
    chunk_g = _chunk_groups(per_worker)
    n_chunks = per_worker // chunk_g
    chunk_rows = chunk_g * _L

    mesh = plsc.VectorSubcoreMesh(
        core_axis_name="c", subcore_axis_name="s",
        num_cores=_NC, num_subcores=_NS)

    @functools.partial(
        pl.kernel,
        out_type=jax.ShapeDtypeStruct((_NW, 64), jnp.int32),
        mesh=mesh,
        interpret=interpret,
        compiler_params=pltpu.CompilerParams(needs_layout_passes=False),
        scratch_types=[
            pltpu.VMEM((chunk_rows * N_CATS,), jnp.float32),
            pltpu.VMEM((chunk_rows * N_CATS,), jnp.float32),
            pltpu.VMEM((chunk_rows,), jnp.int32),
            pltpu.VMEM((chunk_rows,), jnp.int32),
            pltpu.VMEM((_L * 64,), jnp.int32),
            pltpu.VMEM((64,), jnp.int32),
            pltpu.SemaphoreType.DMA,
            pltpu.SemaphoreType.DMA,
            pltpu.SemaphoreType.DMA,
            pltpu.SemaphoreType.DMA,
        ],
    )
    def sc_kernel(inp_hbm, tgt_hbm, out_hbm, inp_v0, inp_v1, tgt_v0, tgt_v1,
                  hist_v, part_v, si0, si1, st0, st1):
        wid = lax.axis_index("s") * _NC + lax.axis_index("c")
        iota = lax.iota(jnp.int32, _L)
        lane_off = iota * 64
        iota8 = iota * N_CATS
        jcols = [jnp.full((_L,), j, jnp.int32) for j in range(N_CATS)]
        ones = jnp.full((_L,), 1, jnp.int32)
        zeros = jnp.full((_L,), 0, jnp.int32)

        for k in range(64):
            hist_v[pl.ds(k * _L, _L)] = zeros

        def group_body(inp_ref, tgt_ref):
            def body(g):
                flat0 = g * (_L * N_CATS) + iota8
                cols = [plsc.load_gather(inp_ref, [flat0 + j])
                        for j in range(N_CATS)]
                bv, bi = [], []
                for j in range(0, N_CATS, 2):
                    gt = cols[j + 1] > cols[j]
                    bv.append(jnp.where(gt, cols[j + 1], cols[j]))
                    bi.append(jnp.where(gt, jcols[j + 1], jcols[j]))
                v0, i0, v1, i1 = bv[0], bi[0], bv[1], bi[1]
                gt = v1 > v0
                va, ia = jnp.where(gt, v1, v0), jnp.where(gt, i1, i0)
                v0, i0, v1, i1 = bv[2], bi[2], bv[3], bi[3]
                gt = v1 > v0
                vb, ib = jnp.where(gt, v1, v0), jnp.where(gt, i1, i0)
                gt = vb > va
                pred = jnp.where(gt, ib, ia)
                tgt = tgt_ref[pl.ds(g * _L, _L)]
                bins = tgt * N_CATS + pred + lane_off
                plsc.addupdate_scatter(hist_v, [bins], ones)
            return body

        row_base = wid * (per_worker * _L)
        bufs = [(inp_v0, tgt_v0, si0, st0), (inp_v1, tgt_v1, si1, st1)]

        def start(c):
            iv, tv, si, st = bufs[c % 2]
            rbase = row_base + c * chunk_rows
            di = pltpu.async_copy(
                inp_hbm.at[pl.ds(rbase * N_CATS, chunk_rows * N_CATS)],
                iv, si)
            dt = pltpu.async_copy(tgt_hbm.at[pl.ds(rbase, chunk_rows)], tv, st)
            return di, dt

        pending = start(0)
        for c in range(n_chunks):
            iv, tv, _, _ = bufs[c % 2]
            pending[0].wait()
            pending[1].wait()
            if c + 1 < n_chunks:
                pending = start(c + 1)
            plsc.parallel_loop(0, chunk_g, 1, unroll=8)(group_body(iv, tv))
        if tail_rem > 0:
            @pl.when(wid < tail_rem)
            def _():
                tbase = _NW * per_worker * _L + wid * _L
                pltpu.sync_copy(inp_hbm.at[pl.ds(tbase * N_CATS, _L * N_CATS)],
                                inp_v0.at[pl.ds(0, _L * N_CATS)])
                pltpu.sync_copy(tgt_hbm.at[pl.ds(tbase, _L)],
                                tgt_v0.at[pl.ds(0, _L)])
                group_body(inp_v0, tgt_v0)(0)

        for q in range(4):
            acc = zeros
            for r in range(_L):
                acc = acc + hist_v[pl.ds(r * 64 + q * _L, _L)]
            part_v[pl.ds(q * _L, _L)] = acc
        pltpu.sync_copy(part_v, out_hbm.at[wid])

    return sc_kernel


def _qwk_epilogue(parts_ref, o_ref, *, n):
    x = parts_ref[...].astype(jnp.float32)
    conf = jnp.sum(x, axis=0) * (1.0 / n)
    marg_true = jnp.sum(conf, axis=1, keepdims=True)
    marg_pred = jnp.sum(conf, axis=0, keepdims=True)
    expected = marg_true * marg_pred
    i = lax.broadcasted_iota(jnp.int32, (N_CATS, N_CATS), 0).astype(jnp.float32)
    j = lax.broadcasted_iota(jnp.int32, (N_CATS, N_CATS), 1).astype(jnp.float32)
    w = 1.0 - (i - j) ** 2 / float((N_CATS - 1) ** 2)
    po = jnp.sum(w * conf)
    pe = jnp.sum(w * expected)
    pe = jnp.clip(pe, 0.0, 1.0 - EPS)
    qwk = jnp.where(pe >= 1.0 - EPS, 0.0, (po - pe) / (1.0 - pe + EPS))
    qwk = jnp.clip(qwk, -1.0, 1.0)
    o_ref[...] = jnp.full((1, 1), 1.0 - qwk, jnp.float32)


def kernel(inputs, targets):
    if inputs.ndim > 2:
        inputs = inputs.reshape(-1, inputs.shape[-1])
        targets = targets.reshape(-1)
    n = inputs.shape[0]
    targets = targets.astype(jnp.int32)
    parts = _sc_partial_hist(n)(inputs.reshape(-1), targets)
    out = pl.pallas_call(
        functools.partial(_qwk_epilogue, n=n),
        out_shape=jax.ShapeDtypeStruct((1, 1), jnp.float32),
    )(parts.reshape(_NW, N_CATS, N_CATS))
    return out[0, 0]
